# Initial kernel scaffold; baseline (speedup 1.0000x reference)
import jax
import jax.numpy as jnp
from jax.experimental import pallas as pl


def kernel(x, W, b):
    raise NotImplementedError("write your pallas kernel here")


# speedup vs baseline: 41.4432x; 41.4432x over previous
import functools

import jax
import jax.numpy as jnp
from jax import lax
from jax.experimental import pallas as pl
from jax.experimental.pallas import tpu as pltpu
from jax.experimental.pallas import tpu_sc as plsc

B, C_IN, C_OUT, L, K = 4, 16, 16, 4096, 4
TR = 256
NT = L // TR
BIG = 3.0e38

_NC, _NS = 2, 16
_NW = _NC * _NS
_TPW = (B * L) // _NW
_IPW = _TPW * K
_ICH = 128
_NCH = _IPW // _ICH


def _topk_body(x_rows_ref, x_all_ref, idx_ref):
    b = pl.program_id(0)
    xr = x_rows_ref[0]
    xa = x_all_ref[0]
    dot = lax.dot_general(xr, xa, (((0,), (0,)), ((), ())),
                          preferred_element_type=jnp.float32)
    na = jnp.sum(xa * xa, axis=0, keepdims=True)
    ones = jnp.ones((C_IN, 1), dtype=jnp.float32)
    nr = lax.dot_general(xr * xr, ones, (((0,), (0,)), ((), ())),
                         preferred_element_type=jnp.float32)
    d = (na + nr) - 2.0 * dot
    d = jnp.maximum(d, 0.0)

    colf = lax.broadcasted_iota(jnp.int32, (TR, L), 1).astype(jnp.float32)
    iks = []
    for k in range(K):
        m = jnp.min(d, axis=1, keepdims=True)
        ikf = jnp.min(jnp.where(d == m, colf, float(L)),
                      axis=1, keepdims=True)
        iks.append(ikf)
        if k < K - 1:
            d = jnp.where(colf == ikf, BIG, d)

    col4 = lax.broadcasted_iota(jnp.int32, (TR, K), 1)
    idxs = jnp.where(col4 == 0, iks[0],
                     jnp.where(col4 == 1, iks[1],
                               jnp.where(col4 == 2, iks[2], iks[3])))
    idx_ref[0] = idxs.astype(jnp.int32) + col4 * L + b * (K * L)


def _topk(x):
    return pl.pallas_call(
        _topk_body,
        grid=(B, NT),
        in_specs=[
            pl.BlockSpec((1, C_IN, TR), lambda b, i: (b, 0, i)),
            pl.BlockSpec((1, C_IN, L), lambda b, i: (b, 0, 0)),
        ],
        out_specs=pl.BlockSpec((1, TR, K), lambda b, i: (b * NT + i, 0, 0)),
        out_shape=jax.ShapeDtypeStruct((B * NT, TR, K), jnp.int32),
    )(x, x)


def _ytable_body(x_ref, w_ref, bias_ref, out_ref):
    xa = x_ref[0]
    w = w_ref[0]
    yt = lax.dot_general(xa, w, (((0,), (1,)), ((), ())),
                         preferred_element_type=jnp.float32)
    out_ref[0, 0] = yt + bias_ref[...][None, :] * (1.0 / K)


def _ytable(x, W, bias):
    Wt = W.transpose(2, 0, 1)
    return pl.pallas_call(
        _ytable_body,
        grid=(B, K),
        in_specs=[
            pl.BlockSpec((1, C_IN, L), lambda b, k: (b, 0, 0)),
            pl.BlockSpec((1, C_OUT, C_IN), lambda b, k: (k, 0, 0)),
            pl.BlockSpec((C_OUT,), lambda b, k: (0,)),
        ],
        out_specs=pl.BlockSpec((1, 1, L, C_OUT), lambda b, k: (b, k, 0, 0)),
        out_shape=jax.ShapeDtypeStruct((B, K, L, C_OUT), jnp.float32),
    )(x, Wt, bias)


def _gather_body(table_hbm, idx_hbm, out_hbm, idx_v, rows_v, out_v, sem):
    c = lax.axis_index("c")
    s = lax.axis_index("s")
    wid = s * _NC + c
    pltpu.sync_copy(idx_hbm.at[pl.ds(wid * _NCH, _NCH)], idx_v)
    copies = [
        pltpu.async_copy(table_hbm.at[idx_v.at[j]],
                         rows_v.at[pl.ds(j * _ICH, _ICH)], sem)
        for j in range(_NCH)
    ]
    for cp in copies:
        cp.wait()

    def body(t, carry):
        base = t * K
        r = ((rows_v[base] + rows_v[base + 1])
             + (rows_v[base + 2] + rows_v[base + 3]))
        out_v[t] = r
        return carry

    lax.fori_loop(0, _TPW, body, 0)
    pltpu.sync_copy(out_v, out_hbm.at[pl.ds(wid * _TPW, _TPW)])


def _gather_sum(table, idx_flat2d):
    mesh = plsc.VectorSubcoreMesh(core_axis_name="c", subcore_axis_name="s")
    run = functools.partial(
        pl.kernel,
        out_type=jax.ShapeDtypeStruct((B * L, C_OUT), jnp.float32),
        mesh=mesh,
        scratch_types=[
            pltpu.VMEM((_NCH, _ICH), jnp.int32),
            pltpu.VMEM((_IPW, C_OUT), jnp.float32),
            pltpu.VMEM((_TPW, C_OUT), jnp.float32),
            pltpu.SemaphoreType.DMA,
        ],
        compiler_params=pltpu.CompilerParams(use_tc_tiling_on_sc=False),
    )(_gather_body)
    return run(table, idx_flat2d)


def kernel(x, W, b):
    idx = _topk(x)
    yt = _ytable(x, W, b)
    table = yt.reshape(B * K * L, C_OUT)
    idx2d = idx.reshape((B * L * K) // _ICH, _ICH)
    out_flat = _gather_sum(table, idx2d)
    return out_flat.reshape(B, L, C_OUT).transpose(0, 2, 1)
